# Initial kernel scaffold; baseline (speedup 1.0000x reference)
import jax
import jax.numpy as jnp
from jax.experimental import pallas as pl


def kernel(x, edge_index, W1, b1, W2, b2):
    raise NotImplementedError("write your pallas kernel here")


# speedup vs baseline: 3.5773x; 3.5773x over previous
import functools

import jax
import jax.numpy as jnp
from jax import lax
from jax.experimental import pallas as pl
from jax.experimental.pallas import tpu as pltpu
from jax.experimental.pallas import tpu_sc as plsc

NC = 2
NS = 16
LANES = 16

CHUNK = 128
ZROWS = 40
RT_ROWS = 1000


def _pq_body(x_ref, wa_ref, wb_ref, b1_ref, p_ref, q_ref):
    xb = x_ref[...]
    wa = wa_ref[0]
    wb = wb_ref[0]
    a = wa - wb
    p_ref[0] = jnp.dot(xb, a, preferred_element_type=jnp.float32) + b1_ref[0]
    q_ref[0] = jnp.dot(xb, wb, preferred_element_type=jnp.float32)


def _compute_pq(x, w1a_h, w1b_h, b1_h, n, d, hh, blk):
    nb = n // blk
    grid = (2, nb)
    return pl.pallas_call(
        _pq_body,
        grid=grid,
        in_specs=[
            pl.BlockSpec((blk, d), lambda h, i: (i, 0)),
            pl.BlockSpec((1, d, hh), lambda h, i: (h, 0, 0)),
            pl.BlockSpec((1, d, hh), lambda h, i: (h, 0, 0)),
            pl.BlockSpec((1, 1, hh), lambda h, i: (h, 0, 0)),
        ],
        out_specs=[
            pl.BlockSpec((1, blk, hh), lambda h, i: (h, i, 0)),
            pl.BlockSpec((1, blk, hh), lambda h, i: (h, i, 0)),
        ],
        out_shape=[
            jax.ShapeDtypeStruct((2, n, hh), jnp.float32),
            jax.ShapeDtypeStruct((2, n, hh), jnp.float32),
        ],
    )(x, w1a_h, w1b_h, b1_h)


def _sc_edge_stage(p_flat, q_flat, dst, src, n, hh, e):
    total_chunks = e // CHUNK
    rt = n // RT_ROWS
    mesh = plsc.VectorSubcoreMesh(core_axis_name="c", subcore_axis_name="s")

    @functools.partial(
        pl.kernel,
        out_type=[
            jax.ShapeDtypeStruct((2 * n, hh), jnp.float32),
            jax.ShapeDtypeStruct((n,), jnp.float32),
        ],
        mesh=mesh,
        scratch_types=[
            pltpu.VMEM_SHARED((n, hh), jnp.float32),
            pltpu.VMEM_SHARED((n,), jnp.float32),
            pltpu.VMEM((CHUNK,), jnp.int32),
            pltpu.VMEM((CHUNK,), jnp.int32),
            pltpu.VMEM((CHUNK,), jnp.int32),
            pltpu.VMEM((CHUNK, hh), jnp.float32),
            pltpu.VMEM((CHUNK,), jnp.float32),
            pltpu.VMEM((ZROWS, hh), jnp.float32),
            pltpu.VMEM((RT_ROWS,), jnp.float32),
            pltpu.SemaphoreType.DMA,
            pltpu.SemaphoreType.DMA,
        ],
    )
    def edge_kernel(p_hbm, q_hbm, dst_hbm, src_hbm, s_out, cnt_out,
                    s_acc, c_acc, dst_v, dsto_v, srco_v, pq_v, ones_v,
                    stage_v, cstage_v, sem_p, sem_q):
        c = lax.axis_index("c")
        s = lax.axis_index("s")
        zero16 = jnp.zeros((LANES,), jnp.float32)
        one16 = jnp.ones((LANES,), jnp.float32)
        hvecs = hh // LANES

        def fill_zero_row(i, _):
            for j in range(hvecs):
                stage_v[i, pl.ds(j * LANES, LANES)] = zero16
            return _
        lax.fori_loop(0, ZROWS, fill_zero_row, None)

        for j in range(CHUNK // LANES):
            ones_v[pl.ds(j * LANES, LANES)] = one16

        def fill_cz(i, _):
            cstage_v[pl.ds(i * LANES, LANES)] = zero16
            return _
        lax.fori_loop(0, RT_ROWS // LANES, fill_cz, None)
        if RT_ROWS % LANES:
            cstage_v[pl.ds(RT_ROWS - LANES, LANES)] = zero16

        @pl.when(s < rt)
        def _():
            for r in range(RT_ROWS // ZROWS):
                pltpu.sync_copy(
                    stage_v, s_acc.at[pl.ds(s * RT_ROWS + r * ZROWS, ZROWS)])
            pltpu.sync_copy(cstage_v, c_acc.at[pl.ds(s * RT_ROWS, RT_ROWS)])
        plsc.subcore_barrier()

        base_chunks = total_chunks // NS
        rem = total_chunks % NS
        nloc = jnp.where(s < rem, base_chunks + 1, base_chunks)
        row_off = c * n

        def chunk_body(i, _):
            base = (i * NS + s) * CHUNK
            pltpu.sync_copy(dst_hbm.at[pl.ds(base, CHUNK)], dst_v)
            pltpu.sync_copy(src_hbm.at[pl.ds(base, CHUNK)], srco_v)
            for j in range(CHUNK // LANES):
                sl = pl.ds(j * LANES, LANES)
                dsto_v[sl] = dst_v[sl] + row_off
                srco_v[sl] = srco_v[sl] + row_off
            pltpu.async_copy(p_hbm.at[dsto_v], pq_v, sem_p).wait()
            pltpu.async_copy(q_hbm.at[srco_v], pq_v, sem_q, add=True).wait()

            def relu_row(r2, _r):
                for j in range(hvecs):
                    sl = pl.ds(j * LANES, LANES)
                    pq_v[r2, sl] = jnp.maximum(pq_v[r2, sl], 0.0)
                return _r
            lax.fori_loop(0, CHUNK, relu_row, None)

            pltpu.sync_copy(pq_v, s_acc.at[dst_v], add=True)

            @pl.when(c == 0)
            def _():
                pltpu.sync_copy(ones_v, c_acc.at[dst_v], add=True)
            return _

        lax.fori_loop(0, nloc, chunk_body, None)
        plsc.subcore_barrier()

        @pl.when(s < rt)
        def _():
            for r in range(RT_ROWS // ZROWS):
                row = s * RT_ROWS + r * ZROWS
                pltpu.sync_copy(s_acc.at[pl.ds(row, ZROWS)], stage_v)
                pltpu.sync_copy(stage_v, s_out.at[pl.ds(row_off + row, ZROWS)])

        @pl.when(jnp.logical_and(c == 0, s < rt))
        def _():
            pltpu.sync_copy(c_acc.at[pl.ds(s * RT_ROWS, RT_ROWS)], cstage_v)
            pltpu.sync_copy(cstage_v, cnt_out.at[pl.ds(s * RT_ROWS, RT_ROWS)])

    return edge_kernel(p_flat, q_flat, dst, src)


def _out_body(s0_ref, s1_ref, cnt_ref, w2_ref, b2_ref, o_ref):
    cntv = cnt_ref[...]
    inv = 1.0 / jnp.maximum(cntv, 1.0)
    h0 = s0_ref[0] * inv
    h1 = s1_ref[0] * inv
    o = (jnp.dot(h0, w2_ref[0], preferred_element_type=jnp.float32)
         + jnp.dot(h1, w2_ref[1], preferred_element_type=jnp.float32)
         + jnp.where(cntv > 0.0, b2_ref[...], 0.0))
    o_ref[...] = jnp.maximum(o, 0.0)


def _compute_out(s_halves, cnt_col, w2_h, b2r, n, d, hh, blk):
    nb = n // blk
    return pl.pallas_call(
        _out_body,
        grid=(nb,),
        in_specs=[
            pl.BlockSpec((1, blk, hh), lambda i: (0, i, 0)),
            pl.BlockSpec((1, blk, hh), lambda i: (1, i, 0)),
            pl.BlockSpec((blk, 1), lambda i: (i, 0)),
            pl.BlockSpec((2, hh, d), lambda i: (0, 0, 0)),
            pl.BlockSpec((1, d), lambda i: (0, 0)),
        ],
        out_specs=pl.BlockSpec((blk, d), lambda i: (i, 0)),
        out_shape=jax.ShapeDtypeStruct((n, d), jnp.float32),
    )(s_halves, s_halves, cnt_col, w2_h, b2r)


def kernel(x, edge_index, W1, b1, W2, b2):
    n, d = x.shape
    e = edge_index.shape[1]
    hh = d
    blk = 400

    src = edge_index[0].astype(jnp.int32)
    dst = edge_index[1].astype(jnp.int32)

    w1t = W1.T.astype(jnp.float32)
    w1a_h = w1t[:d].reshape(d, 2, hh).transpose(1, 0, 2)
    w1b_h = w1t[d:].reshape(d, 2, hh).transpose(1, 0, 2)
    b1_h = b1.astype(jnp.float32).reshape(2, 1, hh)

    p3, q3 = _compute_pq(x.astype(jnp.float32), w1a_h, w1b_h, b1_h,
                         n, d, hh, blk)
    p_flat = p3.reshape(2 * n, hh)
    q_flat = q3.reshape(2 * n, hh)

    s_flat, cnt = _sc_edge_stage(p_flat, q_flat, dst, src, n, hh, e)
    s_halves = s_flat.reshape(2, n, hh)
    cnt_col = cnt.reshape(n, 1)

    w2_h = W2.T.astype(jnp.float32).reshape(2, hh, d)
    b2r = b2.astype(jnp.float32).reshape(1, d)

    return _compute_out(s_halves, cnt_col, w2_h, b2r, n, d, hh, blk)


# speedup vs baseline: 5.7958x; 1.6202x over previous
import functools

import jax
import jax.numpy as jnp
from jax import lax
from jax.experimental import pallas as pl
from jax.experimental.pallas import tpu as pltpu
from jax.experimental.pallas import tpu_sc as plsc

NC = 2
NS = 16
LANES = 16

CHUNK = 128
ZROWS = 40
RT_ROWS = 1000


def _pq_body(x_ref, wa_ref, wb_ref, b1_ref, p_ref, q_ref):
    xb = x_ref[...]
    wa = wa_ref[0]
    wb = wb_ref[0]
    a = wa - wb
    p_ref[0] = jnp.dot(xb, a, preferred_element_type=jnp.float32) + b1_ref[0]
    q_ref[0] = jnp.dot(xb, wb, preferred_element_type=jnp.float32)


def _compute_pq(x, w1a_h, w1b_h, b1_h, n, d, hh, blk):
    nb = n // blk
    grid = (2, nb)
    return pl.pallas_call(
        _pq_body,
        grid=grid,
        in_specs=[
            pl.BlockSpec((blk, d), lambda h, i: (i, 0)),
            pl.BlockSpec((1, d, hh), lambda h, i: (h, 0, 0)),
            pl.BlockSpec((1, d, hh), lambda h, i: (h, 0, 0)),
            pl.BlockSpec((1, 1, hh), lambda h, i: (h, 0, 0)),
        ],
        out_specs=[
            pl.BlockSpec((1, blk, hh), lambda h, i: (h, i, 0)),
            pl.BlockSpec((1, blk, hh), lambda h, i: (h, i, 0)),
        ],
        out_shape=[
            jax.ShapeDtypeStruct((2, n, hh), jnp.float32),
            jax.ShapeDtypeStruct((2, n, hh), jnp.float32),
        ],
    )(x, w1a_h, w1b_h, b1_h)


def _sc_edge_stage(p_flat, q_flat, dst, src, n, hh, e):
    total_chunks = e // CHUNK
    rt = n // RT_ROWS
    mesh = plsc.VectorSubcoreMesh(core_axis_name="c", subcore_axis_name="s")

    @functools.partial(
        pl.kernel,
        out_type=[
            jax.ShapeDtypeStruct((2 * n, hh), jnp.float32),
            jax.ShapeDtypeStruct((2 * n,), jnp.float32),
        ],
        mesh=mesh,
        scratch_types=[
            pltpu.VMEM_SHARED((n, hh), jnp.float32),
            pltpu.VMEM_SHARED((n,), jnp.float32),
            pltpu.VMEM((CHUNK,), jnp.int32),
            pltpu.VMEM((CHUNK,), jnp.int32),
            pltpu.VMEM((CHUNK,), jnp.int32),
            pltpu.VMEM((CHUNK,), jnp.int32),
            pltpu.VMEM((CHUNK,), jnp.int32),
            pltpu.VMEM((CHUNK,), jnp.int32),
            pltpu.VMEM((CHUNK, hh), jnp.float32),
            pltpu.VMEM((CHUNK, hh), jnp.float32),
            pltpu.VMEM((CHUNK,), jnp.float32),
            pltpu.VMEM((ZROWS, hh), jnp.float32),
            pltpu.VMEM((RT_ROWS,), jnp.float32),
            pltpu.SemaphoreType.DMA,
            pltpu.SemaphoreType.DMA,
            pltpu.SemaphoreType.DMA,
            pltpu.SemaphoreType.DMA,
        ],
    )
    def edge_kernel(p_hbm, q_hbm, dst_hbm, src_hbm, s_out, cnt_out,
                    s_acc, c_acc, dst_v0, dst_v1, dsto_v0, dsto_v1,
                    srco_v0, srco_v1, pq_v0, pq_v1, ones_v,
                    stage_v, cstage_v, sem_p0, sem_p1, sem_q0, sem_q1):
        dst_b = (dst_v0, dst_v1)
        dsto_b = (dsto_v0, dsto_v1)
        srco_b = (srco_v0, srco_v1)
        pq_b = (pq_v0, pq_v1)
        sem_p = (sem_p0, sem_p1)
        sem_q = (sem_q0, sem_q1)
        c = lax.axis_index("c")
        s = lax.axis_index("s")
        zero16 = jnp.zeros((LANES,), jnp.float32)
        one16 = jnp.ones((LANES,), jnp.float32)
        hvecs = hh // LANES

        def fill_zero_row(i, _):
            for j in range(hvecs):
                stage_v[i, pl.ds(j * LANES, LANES)] = zero16
            return _
        lax.fori_loop(0, ZROWS, fill_zero_row, None)

        for j in range(CHUNK // LANES):
            ones_v[pl.ds(j * LANES, LANES)] = one16

        def fill_cz(i, _):
            cstage_v[pl.ds(i * LANES, LANES)] = zero16
            return _
        lax.fori_loop(0, RT_ROWS // LANES, fill_cz, None)
        if RT_ROWS % LANES:
            cstage_v[pl.ds(RT_ROWS - LANES, LANES)] = zero16

        @pl.when(s < rt)
        def _():
            for r in range(RT_ROWS // ZROWS):
                pltpu.sync_copy(
                    stage_v, s_acc.at[pl.ds(s * RT_ROWS + r * ZROWS, ZROWS)])
            pltpu.sync_copy(cstage_v, c_acc.at[pl.ds(s * RT_ROWS, RT_ROWS)])
        plsc.subcore_barrier()

        base_chunks = total_chunks // NS
        rem = total_chunks % NS
        nloc = jnp.where(s < rem, base_chunks + 1, base_chunks)
        row_off = c * n

        def load_idx_issue_p(i, b):
            base = (i * NS + s) * CHUNK
            pltpu.sync_copy(dst_hbm.at[pl.ds(base, CHUNK)], dst_b[b])
            pltpu.sync_copy(src_hbm.at[pl.ds(base, CHUNK)], srco_b[b])
            for j in range(CHUNK // LANES):
                sl = pl.ds(j * LANES, LANES)
                dsto_b[b][sl] = dst_b[b][sl] + row_off
                srco_b[b][sl] = srco_b[b][sl] + row_off
            pltpu.async_copy(p_hbm.at[dsto_b[b]], pq_b[b], sem_p[b])

        load_idx_issue_p(0, 0)

        def pair_body(pair, _):
            for b in range(2):
                i = pair * 2 + b

                @pl.when(i < nloc)
                def _():
                    pltpu.make_async_copy(
                        p_hbm.at[dsto_b[b]], pq_b[b], sem_p[b]).wait()
                    pltpu.async_copy(
                        q_hbm.at[srco_b[b]], pq_b[b], sem_q[b], add=True)

                    @pl.when(i + 1 < nloc)
                    def _():
                        load_idx_issue_p(i + 1, 1 - b)

                    pltpu.make_async_copy(
                        q_hbm.at[srco_b[b]], pq_b[b], sem_q[b]).wait()

                    def relu_row(r2, _r):
                        for j in range(hvecs):
                            sl = pl.ds(j * LANES, LANES)
                            pq_b[b][r2, sl] = jnp.maximum(pq_b[b][r2, sl], 0.0)
                        return _r
                    lax.fori_loop(0, CHUNK, relu_row, None)

                    pltpu.sync_copy(pq_b[b], s_acc.at[dst_b[b]], add=True)

                    @pl.when((i % 2) == c)
                    def _():
                        pltpu.sync_copy(ones_v, c_acc.at[dst_b[b]], add=True)
            return _

        lax.fori_loop(0, (nloc + 1) // 2, pair_body, None)
        plsc.subcore_barrier()

        @pl.when(s < rt)
        def _():
            for r in range(RT_ROWS // ZROWS):
                row = s * RT_ROWS + r * ZROWS
                pltpu.sync_copy(s_acc.at[pl.ds(row, ZROWS)], stage_v)
                pltpu.sync_copy(stage_v, s_out.at[pl.ds(row_off + row, ZROWS)])

        @pl.when(s < rt)
        def _():
            pltpu.sync_copy(c_acc.at[pl.ds(s * RT_ROWS, RT_ROWS)], cstage_v)
            pltpu.sync_copy(
                cstage_v, cnt_out.at[pl.ds(c * n + s * RT_ROWS, RT_ROWS)])

    return edge_kernel(p_flat, q_flat, dst, src)


def _out_body(s0_ref, s1_ref, c0_ref, c1_ref, w2_ref, b2_ref, o_ref):
    cntv = c0_ref[0] + c1_ref[0]
    inv = 1.0 / jnp.maximum(cntv, 1.0)
    h0 = s0_ref[0] * inv
    h1 = s1_ref[0] * inv
    o = (jnp.dot(h0, w2_ref[0], preferred_element_type=jnp.float32)
         + jnp.dot(h1, w2_ref[1], preferred_element_type=jnp.float32)
         + jnp.where(cntv > 0.0, b2_ref[...], 0.0))
    o_ref[...] = jnp.maximum(o, 0.0)


def _compute_out(s_halves, cnt_col, w2_h, b2r, n, d, hh, blk):
    nb = n // blk
    return pl.pallas_call(
        _out_body,
        grid=(nb,),
        in_specs=[
            pl.BlockSpec((1, blk, hh), lambda i: (0, i, 0)),
            pl.BlockSpec((1, blk, hh), lambda i: (1, i, 0)),
            pl.BlockSpec((1, blk, 1), lambda i: (0, i, 0)),
            pl.BlockSpec((1, blk, 1), lambda i: (1, i, 0)),
            pl.BlockSpec((2, hh, d), lambda i: (0, 0, 0)),
            pl.BlockSpec((1, d), lambda i: (0, 0)),
        ],
        out_specs=pl.BlockSpec((blk, d), lambda i: (i, 0)),
        out_shape=jax.ShapeDtypeStruct((n, d), jnp.float32),
    )(s_halves, s_halves, cnt_col, cnt_col, w2_h, b2r)


def kernel(x, edge_index, W1, b1, W2, b2):
    n, d = x.shape
    e = edge_index.shape[1]
    hh = d
    blk = 400

    src = edge_index[0].astype(jnp.int32)
    dst = edge_index[1].astype(jnp.int32)

    w1t = W1.T.astype(jnp.float32)
    w1a_h = w1t[:d].reshape(d, 2, hh).transpose(1, 0, 2)
    w1b_h = w1t[d:].reshape(d, 2, hh).transpose(1, 0, 2)
    b1_h = b1.astype(jnp.float32).reshape(2, 1, hh)

    p3, q3 = _compute_pq(x.astype(jnp.float32), w1a_h, w1b_h, b1_h,
                         n, d, hh, blk)
    p_flat = p3.reshape(2 * n, hh)
    q_flat = q3.reshape(2 * n, hh)

    s_flat, cnt = _sc_edge_stage(p_flat, q_flat, dst, src, n, hh, e)
    s_halves = s_flat.reshape(2, n, hh)
    cnt_col = cnt.reshape(2, n, 1)

    w2_h = W2.T.astype(jnp.float32).reshape(2, hh, d)
    b2r = b2.astype(jnp.float32).reshape(1, d)

    return _compute_out(s_halves, cnt_col, w2_h, b2r, n, d, hh, blk)
